# baseline (speedup 1.0000x reference)
import functools
import math

import jax
import jax.numpy as jnp
from jax import lax
from jax.experimental import pallas as pl
from jax.experimental.pallas import tpu as pltpu
from jax.experimental.pallas import tpu_sc as plsc

DIM = 64
SCALE = math.sqrt(DIM)

NC = 2
NS = 16
NW = NC * NS

MB = 8
SEQ_PAD = 56
NBUF = 2


def _pad_body(x_ref, o_ref):
    blk = x_ref.shape[0]
    o_ref[...] = jnp.concatenate(
        [x_ref[...], jnp.zeros((blk, DIM - x_ref.shape[1]), jnp.int32)],
        axis=1,
    )


def _scale_body(seq, i_ref, o_ref):
    o_ref[...] = i_ref[:, :seq, :] * SCALE


def _gather_body(x_hbm, table_hbm, out_hbm, idx_v, rows_v, gsem, ssem):
    bsz = x_hbm.shape[0]
    rows_per_w = bsz // NW
    macros = rows_per_w // MB

    wid = lax.axis_index("s") * NC + lax.axis_index("c")
    brow0 = wid * rows_per_w

    def stage_and_fire(m, b):
        brow = brow0 + m * MB
        pltpu.sync_copy(
            x_hbm.at[pl.ds(brow, MB), pl.ds(0, SEQ_PAD)], idx_v.at[b]
        )
        for j in range(MB):
            pltpu.async_copy(
                table_hbm.at[idx_v.at[b, j]],
                rows_v.at[b, j],
                gsem[b],
            )

    def drain_store(m, b):
        brow = brow0 + m * MB
        for j in range(MB):
            pltpu.make_async_copy(
                table_hbm.at[idx_v.at[b, j]],
                rows_v.at[b, j],
                gsem[b],
            ).wait()
        pltpu.async_copy(rows_v.at[b], out_hbm.at[pl.ds(brow, MB)], ssem[b])

    def wait_store(m, b):
        brow = brow0 + m * MB
        pltpu.make_async_copy(
            rows_v.at[b], out_hbm.at[pl.ds(brow, MB)], ssem[b]
        ).wait()

    stage_and_fire(0, 0)

    @pl.loop(0, macros, step=NBUF)
    def _macro(m0):
        for b in range(NBUF):
            m = m0 + b
            nxt = m + 1
            nb = (b + 1) % NBUF

            @pl.when(nxt < macros)
            def _fire_next():
                @pl.when(m >= 1)
                def _():
                    wait_store(m - 1, nb)
                stage_and_fire(nxt, nb)

            drain_store(m, b)

    wait_store(macros - 1, (macros - 1) % NBUF)


def kernel(x, table):
    bsz, seq = x.shape

    pad_blk = 2048
    xp = pl.pallas_call(
        _pad_body,
        out_shape=jax.ShapeDtypeStruct((bsz, DIM), jnp.int32),
        grid=(bsz // pad_blk,),
        in_specs=[pl.BlockSpec((pad_blk, seq), lambda i: (i, 0))],
        out_specs=pl.BlockSpec((pad_blk, DIM), lambda i: (i, 0)),
    )(x)

    gather_kernel = pl.kernel(
        _gather_body,
        out_type=jax.ShapeDtypeStruct((bsz, SEQ_PAD, DIM), jnp.float32),
        mesh=plsc.VectorSubcoreMesh(
            core_axis_name="c", subcore_axis_name="s",
            num_cores=NC, num_subcores=NS,
        ),
        scratch_types=[
            pltpu.VMEM((NBUF, MB, SEQ_PAD), jnp.int32),
            pltpu.VMEM((NBUF, MB, SEQ_PAD, DIM), jnp.float32),
            [pltpu.SemaphoreType.DMA] * NBUF,
            [pltpu.SemaphoreType.DMA] * NBUF,
        ],
        compiler_params=pltpu.CompilerParams(use_tc_tiling_on_sc=False),
    )
    outp = gather_kernel(xp, table)

    out_blk = 128
    out = pl.pallas_call(
        functools.partial(_scale_body, seq),
        out_shape=jax.ShapeDtypeStruct((bsz, seq, DIM), jnp.float32),
        grid=(bsz // out_blk,),
        in_specs=[pl.BlockSpec((out_blk, SEQ_PAD, DIM), lambda i: (i, 0, 0))],
        out_specs=pl.BlockSpec((out_blk, seq, DIM), lambda i: (i, 0, 0)),
    )(outp)
    return out


# speedup vs baseline: 1.0041x; 1.0041x over previous
import functools
import math

import jax
import jax.numpy as jnp
from jax import lax
from jax.experimental import pallas as pl
from jax.experimental.pallas import tpu as pltpu
from jax.experimental.pallas import tpu_sc as plsc

DIM = 64
SCALE = math.sqrt(DIM)

NC = 2
NS = 16
NW = NC * NS

MB = 8
SEQ_PAD = 56
NBUF = 2


def _pad_body(x_ref, o_ref):
    blk = x_ref.shape[0]
    o_ref[...] = jnp.concatenate(
        [x_ref[...], jnp.zeros((blk, DIM - x_ref.shape[1]), jnp.int32)],
        axis=1,
    )


def _scale_body(seq, i_ref, o_ref):
    blk = o_ref.shape[0]
    rows = i_ref[...].reshape(blk, SEQ_PAD, DIM)
    o_ref[...] = rows[:, :seq, :] * SCALE


def _gather_body(x_hbm, table_hbm, out_hbm, idx_v, rows_v, gsem, ssem):
    bsz = x_hbm.shape[0]
    rows_per_w = bsz // NW
    macros = rows_per_w // MB

    wid = lax.axis_index("s") * NC + lax.axis_index("c")
    brow0 = wid * rows_per_w

    def stage_and_fire(m, b):
        brow = brow0 + m * MB
        pltpu.sync_copy(
            x_hbm.at[pl.ds(brow, MB), pl.ds(0, SEQ_PAD)], idx_v.at[b]
        )
        for j in range(MB):
            pltpu.async_copy(
                table_hbm.at[idx_v.at[b, j]],
                rows_v.at[b, pl.ds(j * SEQ_PAD, SEQ_PAD)],
                gsem[b],
            )

    def drain_store(m, b):
        brow = brow0 + m * MB
        for j in range(MB):
            pltpu.make_async_copy(
                table_hbm.at[idx_v.at[b, j]],
                rows_v.at[b, pl.ds(j * SEQ_PAD, SEQ_PAD)],
                gsem[b],
            ).wait()
        pltpu.async_copy(
            rows_v.at[b],
            out_hbm.at[pl.ds(brow * SEQ_PAD, MB * SEQ_PAD)],
            ssem[b],
        )

    def wait_store(m, b):
        brow = brow0 + m * MB
        pltpu.make_async_copy(
            rows_v.at[b],
            out_hbm.at[pl.ds(brow * SEQ_PAD, MB * SEQ_PAD)],
            ssem[b],
        ).wait()

    stage_and_fire(0, 0)

    @pl.loop(0, macros, step=NBUF)
    def _macro(m0):
        for b in range(NBUF):
            m = m0 + b
            nxt = m + 1
            nb = (b + 1) % NBUF

            @pl.when(nxt < macros)
            def _fire_next():
                @pl.when(m >= 1)
                def _():
                    wait_store(m - 1, nb)
                stage_and_fire(nxt, nb)

            drain_store(m, b)

    wait_store(macros - 1, (macros - 1) % NBUF)


def kernel(x, table):
    bsz, seq = x.shape

    pad_blk = 2048
    xp = pl.pallas_call(
        _pad_body,
        out_shape=jax.ShapeDtypeStruct((bsz, DIM), jnp.int32),
        grid=(bsz // pad_blk,),
        in_specs=[pl.BlockSpec((pad_blk, seq), lambda i: (i, 0))],
        out_specs=pl.BlockSpec((pad_blk, DIM), lambda i: (i, 0)),
    )(x)

    gather_kernel = pl.kernel(
        _gather_body,
        out_type=jax.ShapeDtypeStruct((bsz * SEQ_PAD, DIM), jnp.float32),
        mesh=plsc.VectorSubcoreMesh(
            core_axis_name="c", subcore_axis_name="s",
            num_cores=NC, num_subcores=NS,
        ),
        scratch_types=[
            pltpu.VMEM((NBUF, MB, SEQ_PAD), jnp.int32),
            pltpu.VMEM((NBUF, MB * SEQ_PAD, DIM), jnp.float32),
            [pltpu.SemaphoreType.DMA] * NBUF,
            [pltpu.SemaphoreType.DMA] * NBUF,
        ],
        compiler_params=pltpu.CompilerParams(use_tc_tiling_on_sc=False),
    )
    outp = gather_kernel(xp, table)

    out_blk = 128
    out = pl.pallas_call(
        functools.partial(_scale_body, seq),
        out_shape=jax.ShapeDtypeStruct((bsz, seq, DIM), jnp.float32),
        grid=(bsz // out_blk,),
        in_specs=[pl.BlockSpec((out_blk * SEQ_PAD, DIM), lambda i: (i, 0))],
        out_specs=pl.BlockSpec((out_blk, seq, DIM), lambda i: (i, 0, 0)),
    )(outp)
    return out


# speedup vs baseline: 2.2779x; 2.2685x over previous
import functools
import math

import jax
import jax.numpy as jnp
from jax import lax
from jax.experimental import pallas as pl
from jax.experimental.pallas import tpu as pltpu
from jax.experimental.pallas import tpu_sc as plsc

DIM = 64
SCALE = math.sqrt(DIM)

NC = 2
NS = 16
NW = NC * NS

MB = 8
SEQ_PAD = 56
NBUF = 2


def _pad_body(x_ref, o_ref):
    pad = DIM - x_ref.shape[1]
    o_ref[...] = jnp.concatenate([x_ref[...], x_ref[:, :pad]], axis=1)


def _scale_body(seq, i_ref, o_ref):
    blk = o_ref.shape[0]
    rows = i_ref[...].reshape(blk, SEQ_PAD, DIM)
    o_ref[...] = rows[:, :seq, :] * SCALE


def _gather_body(x_hbm, table_hbm, out_hbm, idx_v, rows_v, gsem, ssem):
    bsz = x_hbm.shape[0]
    rows_per_w = bsz // NW
    macros = rows_per_w // MB

    wid = lax.axis_index("s") * NC + lax.axis_index("c")
    brow0 = wid * rows_per_w

    def stage_and_fire(m, b):
        brow = brow0 + m * MB
        pltpu.sync_copy(
            x_hbm.at[pl.ds(brow, MB), pl.ds(0, SEQ_PAD)], idx_v.at[b]
        )
        for j in range(MB):
            pltpu.async_copy(
                table_hbm.at[idx_v.at[b, j]],
                rows_v.at[b, pl.ds(j * SEQ_PAD, SEQ_PAD)],
                gsem[b],
            )

    def drain_store(m, b):
        brow = brow0 + m * MB
        for j in range(MB):
            pltpu.make_async_copy(
                table_hbm.at[idx_v.at[b, j]],
                rows_v.at[b, pl.ds(j * SEQ_PAD, SEQ_PAD)],
                gsem[b],
            ).wait()
        pltpu.async_copy(
            rows_v.at[b],
            out_hbm.at[pl.ds(brow * SEQ_PAD, MB * SEQ_PAD)],
            ssem[b],
        )

    def wait_store(m, b):
        brow = brow0 + m * MB
        pltpu.make_async_copy(
            rows_v.at[b],
            out_hbm.at[pl.ds(brow * SEQ_PAD, MB * SEQ_PAD)],
            ssem[b],
        ).wait()

    stage_and_fire(0, 0)

    @pl.loop(0, macros, step=NBUF)
    def _macro(m0):
        for b in range(NBUF):
            m = m0 + b
            nxt = m + 1
            nb = (b + 1) % NBUF

            @pl.when(nxt < macros)
            def _fire_next():
                @pl.when(m >= 1)
                def _():
                    wait_store(m - 1, nb)
                stage_and_fire(nxt, nb)

            drain_store(m, b)

    wait_store(macros - 1, (macros - 1) % NBUF)


def kernel(x, table):
    bsz, seq = x.shape

    pad_blk = 2048
    xp = pl.pallas_call(
        _pad_body,
        out_shape=jax.ShapeDtypeStruct((bsz, DIM), jnp.int32),
        grid=(bsz // pad_blk,),
        in_specs=[pl.BlockSpec((pad_blk, seq), lambda i: (i, 0))],
        out_specs=pl.BlockSpec((pad_blk, DIM), lambda i: (i, 0)),
    )(x)

    gather_kernel = pl.kernel(
        _gather_body,
        out_type=jax.ShapeDtypeStruct((bsz * SEQ_PAD, DIM), jnp.float32),
        mesh=plsc.VectorSubcoreMesh(
            core_axis_name="c", subcore_axis_name="s",
            num_cores=NC, num_subcores=NS,
        ),
        scratch_types=[
            pltpu.VMEM((NBUF, MB, SEQ_PAD), jnp.int32),
            pltpu.VMEM((NBUF, MB * SEQ_PAD, DIM), jnp.float32),
            [pltpu.SemaphoreType.DMA] * NBUF,
            [pltpu.SemaphoreType.DMA] * NBUF,
        ],
        compiler_params=pltpu.CompilerParams(use_tc_tiling_on_sc=False),
    )
    outp = gather_kernel(xp, table)

    out_blk = 128
    out = pl.pallas_call(
        functools.partial(_scale_body, seq),
        out_shape=jax.ShapeDtypeStruct((bsz, seq, DIM), jnp.float32),
        grid=(bsz // out_blk,),
        in_specs=[pl.BlockSpec((out_blk * SEQ_PAD, DIM), lambda i: (i, 0))],
        out_specs=pl.BlockSpec((out_blk, seq, DIM), lambda i: (i, 0, 0)),
    )(outp)
    return out


# speedup vs baseline: 2.9964x; 1.3155x over previous
import functools
import math

import jax
import jax.numpy as jnp
from jax import lax
from jax.experimental import pallas as pl
from jax.experimental.pallas import tpu as pltpu
from jax.experimental.pallas import tpu_sc as plsc

DIM = 64
SCALE = math.sqrt(DIM)

NC = 2
NS = 16
NW = NC * NS

MB = 8
SEQ_PAD = 56
NBUF = 2


def _pad_body(x_ref, o_ref):
    pad = DIM - x_ref.shape[1]
    o_ref[...] = jnp.concatenate([x_ref[...], x_ref[:, :pad]], axis=1)


def _gather_body(x_hbm, table_hbm, out_hbm, idx_v, rows_v, gsem, ssem):
    bsz = x_hbm.shape[0]
    rows_per_w = bsz // NW
    macros = rows_per_w // MB

    wid = lax.axis_index("s") * NC + lax.axis_index("c")
    brow0 = wid * rows_per_w

    def stage_and_fire(m, b):
        brow = brow0 + m * MB
        pltpu.sync_copy(
            x_hbm.at[pl.ds(brow, MB), pl.ds(0, SEQ_PAD)], idx_v.at[b]
        )
        for j in range(MB):
            pltpu.async_copy(
                table_hbm.at[idx_v.at[b, j]],
                rows_v.at[b, pl.ds(j * SEQ_PAD, SEQ_PAD)],
                gsem[b],
            )

    def drain_store(m, b):
        brow = brow0 + m * MB
        for j in range(MB):
            pltpu.make_async_copy(
                table_hbm.at[idx_v.at[b, j]],
                rows_v.at[b, pl.ds(j * SEQ_PAD, SEQ_PAD)],
                gsem[b],
            ).wait()

        @pl.loop(0, MB * SEQ_PAD, unroll=4)
        def _scale(r):
            for j in range(DIM // 16):
                sl = pl.ds(j * 16, 16)
                rows_v[b, r, sl] = rows_v[b, r, sl] * SCALE

        pltpu.async_copy(
            rows_v.at[b],
            out_hbm.at[pl.ds(brow * SEQ_PAD, MB * SEQ_PAD)],
            ssem[b],
        )

    def wait_store(m, b):
        brow = brow0 + m * MB
        pltpu.make_async_copy(
            rows_v.at[b],
            out_hbm.at[pl.ds(brow * SEQ_PAD, MB * SEQ_PAD)],
            ssem[b],
        ).wait()

    stage_and_fire(0, 0)

    @pl.loop(0, macros, step=NBUF)
    def _macro(m0):
        for b in range(NBUF):
            m = m0 + b
            nxt = m + 1
            nb = (b + 1) % NBUF

            @pl.when(nxt < macros)
            def _fire_next():
                @pl.when(m >= 1)
                def _():
                    wait_store(m - 1, nb)
                stage_and_fire(nxt, nb)

            drain_store(m, b)

    wait_store(macros - 1, (macros - 1) % NBUF)


def kernel(x, table):
    bsz, seq = x.shape

    pad_blk = 2048
    xp = pl.pallas_call(
        _pad_body,
        out_shape=jax.ShapeDtypeStruct((bsz, DIM), jnp.int32),
        grid=(bsz // pad_blk,),
        in_specs=[pl.BlockSpec((pad_blk, seq), lambda i: (i, 0))],
        out_specs=pl.BlockSpec((pad_blk, DIM), lambda i: (i, 0)),
    )(x)

    gather_kernel = pl.kernel(
        _gather_body,
        out_type=jax.ShapeDtypeStruct((bsz * SEQ_PAD, DIM), jnp.float32),
        mesh=plsc.VectorSubcoreMesh(
            core_axis_name="c", subcore_axis_name="s",
            num_cores=NC, num_subcores=NS,
        ),
        scratch_types=[
            pltpu.VMEM((NBUF, MB, SEQ_PAD), jnp.int32),
            pltpu.VMEM((NBUF, MB * SEQ_PAD, DIM), jnp.float32),
            [pltpu.SemaphoreType.DMA] * NBUF,
            [pltpu.SemaphoreType.DMA] * NBUF,
        ],
        compiler_params=pltpu.CompilerParams(use_tc_tiling_on_sc=False),
    )
    outp = gather_kernel(xp, table)
    return outp.reshape(bsz, SEQ_PAD, DIM)[:, :seq, :]
